# baseline (speedup 1.0000x reference)
import functools

import jax
import jax.numpy as jnp
from jax import lax
from jax.experimental import pallas as pl
from jax.experimental.pallas import tpu as pltpu
from jax.experimental.pallas import tpu_sc as plsc

B, H, D = 4096, 26, 64
N_PER_HEAD = 100000
NC, NS, L = 2, 16, 16
NW = NC * NS
TOTAL = B * H
IDX_W = 128
ROWS_PER_W = TOTAL // NW
J_PER_W = ROWS_PER_W // IDX_W


def _sc_body(ids_hbm, pairs_hbm, out_hbm, idx_v, rows_v, sem0, sem1):
    wid = lax.axis_index("c") * NS + lax.axis_index("s")
    base = wid * ROWS_PER_W

    pltpu.sync_copy(ids_hbm.at[pl.ds(base, ROWS_PER_W)], idx_v)

    lanes = lax.iota(jnp.int32, L)

    def shift_vec(t, _):
        pos = base + t * L + lanes
        head = lax.rem(pos, H)
        sl = pl.ds(t * L, L)
        idx_v[sl] = lax.shift_right_logical(idx_v[sl] + head * N_PER_HEAD, 1)
        return 0

    lax.fori_loop(0, ROWS_PER_W // L, shift_vec, 0)

    sems = (sem0, sem1)

    def start(j, b):
        pltpu.make_async_copy(pairs_hbm.at[idx_v.at[pl.ds(j * IDX_W, IDX_W)]],
                              rows_v.at[b], sems[b]).start()

    def drain(j, b):
        pltpu.make_async_copy(pairs_hbm.at[idx_v.at[pl.ds(j * IDX_W, IDX_W)]],
                              rows_v.at[b], sems[b]).wait()
        pltpu.sync_copy(rows_v.at[b], out_hbm.at[pl.ds(base + j * IDX_W, IDX_W)])

    start(0, 0)
    start(1, 1)

    def step(t, _):
        for b in range(2):
            drain(2 * t + b, b)
            start(2 * t + b + 2, b)
        return 0

    lax.fori_loop(0, J_PER_W // 2 - 1, step, 0, unroll=False)
    for b in range(2):
        drain(J_PER_W - 2 + b, b)


@functools.partial(
    pl.kernel,
    out_type=jax.ShapeDtypeStruct((TOTAL, 128), jnp.float32),
    mesh=plsc.VectorSubcoreMesh(core_axis_name="c", subcore_axis_name="s"),
    compiler_params=pltpu.CompilerParams(use_tc_tiling_on_sc=True),
    scratch_types=[
        pltpu.VMEM((ROWS_PER_W,), jnp.int32),
        pltpu.VMEM((2, IDX_W, 128), jnp.float32),
        pltpu.SemaphoreType.DMA,
        pltpu.SemaphoreType.DMA,
    ],
)
def _mhe_gather(ids_hbm, pairs_hbm, out_hbm, idx_v, rows_v, sem0, sem1):
    _sc_body(ids_hbm, pairs_hbm, out_hbm, idx_v, rows_v, sem0, sem1)


def kernel(input_ids, table):
    ids_flat = input_ids.reshape(TOTAL).astype(jnp.int32)
    pairs = table.reshape(TOTAL // TOTAL * 1300000, 128)
    out_pairs = _mhe_gather(ids_flat, pairs)
    odd = (ids_flat & 1).astype(bool)[:, None]
    out = jnp.where(odd, out_pairs[:, 64:], out_pairs[:, :64])
    return out.reshape(B, H, D)


# speedup vs baseline: 1.0629x; 1.0629x over previous
import functools

import jax
import jax.numpy as jnp
from jax import lax
from jax.experimental import pallas as pl
from jax.experimental.pallas import tpu as pltpu
from jax.experimental.pallas import tpu_sc as plsc

B, H, D = 4096, 26, 64
N_PER_HEAD = 100000
NC, NS, L = 2, 16, 16
NW = NC * NS
TOTAL = B * H
IDX_W = 128
ROWS_PER_W = TOTAL // NW
J_PER_W = ROWS_PER_W // IDX_W
NBUF = 4
SKEW = 2


def _sc_body(ids_hbm, table_hbm, out_hbm, idx_v, rows_v, *sems):
    semg, semw = sems[:NBUF], sems[NBUF:]
    wid = lax.axis_index("c") * NS + lax.axis_index("s")
    base = wid * ROWS_PER_W

    pltpu.sync_copy(ids_hbm.at[pl.ds(base, ROWS_PER_W)], idx_v)

    lanes = lax.iota(jnp.int32, L)

    def shift_vec(t, _):
        pos = base + t * L + lanes
        head = lax.rem(pos, H)
        sl = pl.ds(t * L, L)
        idx_v[sl] = idx_v[sl] + head * N_PER_HEAD
        return 0

    lax.fori_loop(0, ROWS_PER_W // L, shift_vec, 0)

    def gather(t):
        b = t % NBUF
        return pltpu.make_async_copy(
            table_hbm.at[idx_v.at[pl.ds(t * IDX_W, IDX_W)]],
            rows_v.at[b], semg[b])

    def scatter(t):
        b = t % NBUF
        return pltpu.make_async_copy(
            rows_v.at[b], out_hbm.at[pl.ds(base + t * IDX_W, IDX_W)], semw[b])

    for t in range(J_PER_W):
        if t >= NBUF:
            scatter(t - NBUF).wait()
        gather(t).start()
        if t >= SKEW:
            gather(t - SKEW).wait()
            scatter(t - SKEW).start()
    for t in range(J_PER_W - SKEW, J_PER_W):
        gather(t).wait()
        scatter(t).start()
    for t in range(J_PER_W - NBUF, J_PER_W):
        scatter(t).wait()


@functools.partial(
    pl.kernel,
    out_type=jax.ShapeDtypeStruct((TOTAL, D), jnp.float32),
    mesh=plsc.VectorSubcoreMesh(core_axis_name="c", subcore_axis_name="s"),
    compiler_params=pltpu.CompilerParams(use_tc_tiling_on_sc=False),
    scratch_types=[
        pltpu.VMEM((ROWS_PER_W,), jnp.int32),
        pltpu.VMEM((NBUF, IDX_W, D), jnp.float32),
    ] + [pltpu.SemaphoreType.DMA] * (2 * NBUF),
)
def _mhe_gather(ids_hbm, table_hbm, out_hbm, idx_v, rows_v, *sems):
    _sc_body(ids_hbm, table_hbm, out_hbm, idx_v, rows_v, *sems)


def kernel(input_ids, table):
    ids_flat = input_ids.reshape(TOTAL).astype(jnp.int32)
    out = _mhe_gather(ids_flat, table)
    return out.reshape(B, H, D)


# speedup vs baseline: 1.1709x; 1.1015x over previous
import functools

import jax
import jax.numpy as jnp
from jax import lax
from jax.experimental import pallas as pl
from jax.experimental.pallas import tpu as pltpu
from jax.experimental.pallas import tpu_sc as plsc

B, H, D = 4096, 26, 64
N_PER_HEAD = 100000
NC, NS, L = 2, 16, 16
NW = NC * NS
TOTAL = B * H
DP = 128
IDX_W = 128
ROWS_PER_W = TOTAL // NW
J_PER_W = ROWS_PER_W // IDX_W
NBUF = 2
SKEW = 1


def _sc_body(ids_hbm, table_hbm, out_hbm, idx_v, rows_v, *sems):
    semg, semw = sems[:NBUF], sems[NBUF:]
    wid = lax.axis_index("c") * NS + lax.axis_index("s")
    base = wid * ROWS_PER_W

    pltpu.sync_copy(ids_hbm.at[pl.ds(base, ROWS_PER_W)], idx_v)

    lanes = lax.iota(jnp.int32, L)

    def shift_vec(t, _):
        pos = base + t * L + lanes
        head = lax.rem(pos, H)
        sl = pl.ds(t * L, L)
        idx_v[sl] = idx_v[sl] + head * N_PER_HEAD
        return 0

    lax.fori_loop(0, ROWS_PER_W // L, shift_vec, 0)

    def gather(t):
        b = t % NBUF
        return pltpu.make_async_copy(
            table_hbm.at[idx_v.at[pl.ds(t * IDX_W, IDX_W)]],
            rows_v.at[b], semg[b])

    def scatter(t):
        b = t % NBUF
        return pltpu.make_async_copy(
            rows_v.at[b], out_hbm.at[pl.ds(base + t * IDX_W, IDX_W)], semw[b])

    for t in range(J_PER_W):
        if t >= NBUF:
            scatter(t - NBUF).wait()
        gather(t).start()
        if t >= SKEW:
            gather(t - SKEW).wait()
            scatter(t - SKEW).start()
    for t in range(J_PER_W - SKEW, J_PER_W):
        gather(t).wait()
        scatter(t).start()
    for t in range(J_PER_W - NBUF, J_PER_W):
        scatter(t).wait()


@functools.partial(
    pl.kernel,
    out_type=jax.ShapeDtypeStruct((TOTAL, DP), jnp.float32),
    mesh=plsc.VectorSubcoreMesh(core_axis_name="c", subcore_axis_name="s"),
    compiler_params=pltpu.CompilerParams(use_tc_tiling_on_sc=True),
    scratch_types=[
        pltpu.VMEM((ROWS_PER_W,), jnp.int32),
        pltpu.VMEM((NBUF, IDX_W, DP), jnp.float32),
    ] + [pltpu.SemaphoreType.DMA] * (2 * NBUF),
)
def _mhe_gather(ids_hbm, table_hbm, out_hbm, idx_v, rows_v, *sems):
    _sc_body(ids_hbm, table_hbm, out_hbm, idx_v, rows_v, *sems)


def kernel(input_ids, table):
    ids_flat = input_ids.reshape(TOTAL).astype(jnp.int32)
    table_p = jnp.pad(table, ((0, 0), (0, DP - D)))
    out = _mhe_gather(ids_flat, table_p)
    return out[:, :D].reshape(B, H, D)
